# baseline (speedup 1.0000x reference)
import functools

import jax
import jax.numpy as jnp
from jax import lax
from jax.experimental import pallas as pl
from jax.experimental.pallas import tpu as pltpu
from jax.experimental.pallas import tpu_sc as plsc

_THRESHOLDS = (1, 2, 3, 4, 5, 6, 7, 8, 12, 16, 23, 32, 46, 64, 91)

_N_HEADS = 16
_Q = 2048
_K = 2048
_DV = 4096
_DA = 3968
_N_SHIFTS = 128
_HPC = _N_HEADS // 2
_NBUF = 4


def _diag_body(delta_ref, emb_ref, out_ref):
    dd = delta_ref[0]
    xg = lax.broadcasted_iota(jnp.int32, (32, _DV), 1)
    bb = lax.broadcasted_iota(jnp.int32, (32, _DV), 0)
    rp = xg - (_Q - 1) + dd
    a = jnp.abs(rp)
    g = jnp.zeros((32, _DV), jnp.int32)
    for t in _THRESHOLDS:
        g = g + (a >= t).astype(jnp.int32)
    bucket = jnp.where(rp > 0, 16, 0) + g
    onehot = (bucket == bb).astype(jnp.float32)
    hh = pl.program_id(0)
    row = lax.dot_general(
        emb_ref[pl.ds(hh, 1), :], onehot,
        dimension_numbers=(((1,), (0,)), ((), ())),
        preferred_element_type=jnp.float32,
        precision=lax.Precision.HIGHEST,
    )
    for s in range(_N_SHIFTS):
        out_ref[0, s, :] = row[0, 127 - s : 127 - s + _DA]


def _build_shifts(delta, emb):
    return pl.pallas_call(
        _diag_body,
        grid=(_N_HEADS,),
        out_shape=jax.ShapeDtypeStruct(
            (_N_HEADS, _N_SHIFTS, _DA), jnp.float32
        ),
        in_specs=[
            pl.BlockSpec(memory_space=pltpu.SMEM),
            pl.BlockSpec((16, 32), lambda h: (0, 0)),
        ],
        out_specs=pl.BlockSpec((1, _N_SHIFTS, _DA), lambda h: (h, 0, 0)),
    )(delta, emb)


def _materialize_body(shifts_hbm, out_hbm, *refs):
    spms = refs[:_NBUF]
    sems = refs[_NBUF:2 * _NBUF]
    sem_w = refs[2 * _NBUF]
    c = lax.axis_index("c")
    sid = lax.axis_index("s")
    h0 = c * _HPC
    start = pl.multiple_of((_Q - 128) - 128 * sid, 128)
    row0 = pl.multiple_of(128 * sid, 8)

    srow = pl.multiple_of(8 * sid, 8)

    def stage(idx):
        return pltpu.make_async_copy(
            shifts_hbm.at[h0 + idx, pl.ds(srow, 8), :],
            spms[idx % _NBUF].at[pl.ds(srow, 8), :],
            sems[idx % _NBUF],
        )

    def write(idx):
        return pltpu.make_async_copy(
            spms[idx % _NBUF].at[:, pl.ds(start, _K)],
            out_hbm.at[h0 + idx, pl.ds(row0, 128), :],
            sem_w,
        )

    stage(0).start()
    stage(1).start()

    for idx in range(_HPC):
        if idx >= _NBUF - 2:
            write(idx - _NBUF + 2).wait()
        plsc.subcore_barrier()
        if idx + 2 < _HPC:
            stage(idx + 2).start()
        stage(idx).wait()
        plsc.subcore_barrier()
        write(idx).start()

    for idx in range(_HPC - _NBUF + 2, _HPC):
        write(idx).wait()


@functools.cache
def _make_materialize():
    mesh = plsc.VectorSubcoreMesh(core_axis_name="c", subcore_axis_name="s")
    return pl.kernel(
        _materialize_body,
        mesh=mesh,
        out_type=jax.ShapeDtypeStruct((_N_HEADS, _Q, _K), jnp.float32),
        scratch_types=(
            [pltpu.VMEM_SHARED((_N_SHIFTS, _DA), jnp.float32)] * _NBUF
            + [pltpu.SemaphoreType.DMA] * (_NBUF + 1)
        ),
    )


def kernel(query_length, key_length, offset, embeddings):
    del offset
    delta = (
        jnp.asarray(key_length, jnp.int32) - jnp.asarray(query_length, jnp.int32)
    ).reshape(1)
    shifts = _build_shifts(delta, embeddings.T)
    return _make_materialize()(shifts)


# speedup vs baseline: 1.0189x; 1.0189x over previous
import functools

import jax
import jax.numpy as jnp
from jax import lax
from jax.experimental import pallas as pl
from jax.experimental.pallas import tpu as pltpu
from jax.experimental.pallas import tpu_sc as plsc

_THRESHOLDS = (1, 2, 3, 4, 5, 6, 7, 8, 12, 16, 23, 32, 46, 64, 91)

_N_HEADS = 16
_Q = 2048
_K = 2048
_DV = 4096
_DA = 3968
_N_SHIFTS = 128
_HPC = _N_HEADS // 2
_NBUF = 4


def _diag_body(delta_ref, emb_ref, out_ref):
    dd = delta_ref[0]
    xg = lax.broadcasted_iota(jnp.int32, (32, _DV), 1)
    bb = lax.broadcasted_iota(jnp.int32, (32, _DV), 0)
    rp = xg - (_Q - 1) + dd
    a = jnp.abs(rp)
    g = jnp.zeros((32, _DV), jnp.int32)
    for t in _THRESHOLDS:
        g = g + (a >= t).astype(jnp.int32)
    bucket = jnp.where(rp > 0, 16, 0) + g
    onehot = (bucket == bb).astype(jnp.float32)
    hh = pl.program_id(0)
    row = lax.dot_general(
        emb_ref[pl.ds(hh, 1), :], onehot,
        dimension_numbers=(((1,), (0,)), ((), ())),
        preferred_element_type=jnp.float32,
        precision=lax.Precision.HIGHEST,
    )
    for s in range(_N_SHIFTS):
        out_ref[0, s, :] = row[0, 127 - s : 127 - s + _DA]


def _build_shifts(delta, emb):
    return pl.pallas_call(
        _diag_body,
        grid=(_N_HEADS,),
        out_shape=jax.ShapeDtypeStruct(
            (_N_HEADS, _N_SHIFTS, _DA), jnp.float32
        ),
        in_specs=[
            pl.BlockSpec(memory_space=pltpu.SMEM),
            pl.BlockSpec((16, 32), lambda h: (0, 0)),
        ],
        out_specs=pl.BlockSpec((1, _N_SHIFTS, _DA), lambda h: (h, 0, 0)),
    )(delta, emb)


def _materialize_body(shifts_hbm, out_hbm, *refs):
    spms = refs[:_NBUF]
    sems = refs[_NBUF:2 * _NBUF]
    sem_w = refs[2 * _NBUF]
    c = lax.axis_index("c")
    sid = lax.axis_index("s")
    h0 = c * _HPC
    start = pl.multiple_of((_Q - 128) - 128 * sid, 128)
    row0 = pl.multiple_of(128 * sid, 8)

    def stage(idx):
        return pltpu.make_async_copy(
            shifts_hbm.at[h0 + idx], spms[idx % _NBUF], sems[idx % _NBUF]
        )

    def write(idx):
        return pltpu.make_async_copy(
            spms[idx % _NBUF].at[:, pl.ds(start, _K)],
            out_hbm.at[h0 + idx, pl.ds(row0, 128), :],
            sem_w,
        )

    @pl.when(sid == 0)
    def _prologue():
        stage(0).start()
        stage(1).start()

    for idx in range(_HPC):
        if idx >= _NBUF - 2:
            write(idx - _NBUF + 2).wait()
        plsc.subcore_barrier()

        @pl.when(sid == 0)
        def _stager(idx=idx):
            if idx + 2 < _HPC:
                stage(idx + 2).start()
            stage(idx).wait()

        plsc.subcore_barrier()
        write(idx).start()

    for idx in range(_HPC - _NBUF + 2, _HPC):
        write(idx).wait()


@functools.cache
def _make_materialize():
    mesh = plsc.VectorSubcoreMesh(core_axis_name="c", subcore_axis_name="s")
    return pl.kernel(
        _materialize_body,
        mesh=mesh,
        out_type=jax.ShapeDtypeStruct((_N_HEADS, _Q, _K), jnp.float32),
        scratch_types=(
            [pltpu.VMEM_SHARED((_N_SHIFTS, _DA), jnp.float32)] * _NBUF
            + [pltpu.SemaphoreType.DMA] * (_NBUF + 1)
        ),
    )


def kernel(query_length, key_length, offset, embeddings):
    del offset
    delta = (
        jnp.asarray(key_length, jnp.int32) - jnp.asarray(query_length, jnp.int32)
    ).reshape(1)
    shifts = _build_shifts(delta, embeddings.T)
    return _make_materialize()(shifts)
